# baseline (speedup 1.0000x reference)
import functools

import jax
import jax.numpy as jnp
from jax import lax
from jax.experimental import pallas as pl
from jax.experimental.pallas import tpu as pltpu
from jax.experimental.pallas import tpu_sc as plsc

_B = 16384
_L = 200
_D = 256
_NBINS = 48
_NW = 32
_ROWS_PER_W = _B // _NW
_BLK = 64
_NBLK = _ROWS_PER_W // _BLK


def _hist_body(s_hbm, c_hbm, o_hbm, out_hbm, s_v, c_v, o_v, cnt_v):
    wid = lax.axis_index("s") * 2 + lax.axis_index("c")
    base = wid * _ROWS_PER_W
    ones = jnp.ones((16,), jnp.float32)
    zeros = jnp.zeros((16,), jnp.float32)
    lane = lax.iota(jnp.int32, 16)
    tail_mask = lane >= 8

    def blk_body(bi, carry):
        row0 = base + bi * _BLK
        pltpu.sync_copy(s_hbm.at[pl.ds(row0, _BLK)], s_v)
        pltpu.sync_copy(c_hbm.at[pl.ds(row0, _BLK)], c_v)
        pltpu.sync_copy(o_hbm.at[pl.ds(row0, _BLK)], o_v)

        def row_body(r, carry2):
            roff = r * _NBINS
            cnt_v[pl.ds(roff, 16)] = zeros
            cnt_v[pl.ds(roff + 16, 16)] = zeros
            cnt_v[pl.ds(roff + 32, 16)] = zeros
            for ids_v, fbase in ((s_v, 0), (c_v, 20), (o_v, 35)):
                for k in range(12):
                    idx = ids_v[r, pl.ds(k * 16, 16)] + (roff + fbase)
                    plsc.addupdate_scatter(cnt_v, [idx], ones)
                idx = ids_v[r, pl.ds(_L - 16, 16)] + (roff + fbase)
                plsc.addupdate_scatter(cnt_v, [idx], ones, mask=tail_mask)
            return carry2

        lax.fori_loop(0, _BLK, row_body, 0)
        pltpu.sync_copy(cnt_v, out_hbm.at[pl.ds(row0 * _NBINS,
                                                _BLK * _NBINS)])
        return carry

    lax.fori_loop(0, _NBLK, blk_body, 0)


@functools.partial(
    pl.kernel,
    out_type=jax.ShapeDtypeStruct((_B * _NBINS,), jnp.float32),
    mesh=plsc.VectorSubcoreMesh(core_axis_name="c", subcore_axis_name="s"),
    compiler_params=pltpu.CompilerParams(needs_layout_passes=False),
    scratch_types=[
        pltpu.VMEM((_BLK, _L), jnp.int32),
        pltpu.VMEM((_BLK, _L), jnp.int32),
        pltpu.VMEM((_BLK, _L), jnp.int32),
        pltpu.VMEM((_BLK * _NBINS,), jnp.float32),
    ],
)
def _sc_histogram(s_hbm, c_hbm, o_hbm, out_hbm, s_v, c_v, o_v, cnt_v):
    _hist_body(s_hbm, c_hbm, o_hbm, out_hbm, s_v, c_v, o_v, cnt_v)


_TILE = 512


def _mlp_kernel_body(cnt_ref, wf_ref, b1_ref, w2_ref, b2_ref, w3_ref, b3_ref,
                     out_ref):
    h = jnp.maximum(
        jnp.dot(cnt_ref[...], wf_ref[...], preferred_element_type=jnp.float32)
        + b1_ref[...], 0.0)
    h = jnp.maximum(
        jnp.dot(h, w2_ref[...], preferred_element_type=jnp.float32)
        + b2_ref[...], 0.0)
    out_ref[...] = (
        jnp.dot(h, w3_ref[...], preferred_element_type=jnp.float32)
        + b3_ref[...])


def kernel(style_ids, color_ids, occasion_ids, style_table, color_table,
           occasion_table, W1, b1, W2, b2, W3, b3):
    b = style_ids.shape[0]
    q = style_table.shape[1]
    wf = jnp.concatenate([
        style_table @ W1[:q],
        color_table @ W1[q:2 * q],
        occasion_table @ W1[2 * q:3 * q],
    ], axis=0) * (1.0 / _L)
    wf = jnp.pad(wf, ((0, _NBINS - wf.shape[0]), (0, 0)))

    counts = _sc_histogram(style_ids, color_ids,
                           occasion_ids).reshape(b, _NBINS)

    grid = (b // _TILE,)
    w_spec = lambda shape: pl.BlockSpec(shape, lambda i: (0,) * len(shape))
    return pl.pallas_call(
        _mlp_kernel_body,
        grid=grid,
        in_specs=[
            pl.BlockSpec((_TILE, _NBINS), lambda i: (i, 0)),
            w_spec((_NBINS, _D)),
            w_spec((1, _D)),
            w_spec((_D, _D)),
            w_spec((1, _D)),
            w_spec((_D, _D)),
            w_spec((1, _D)),
        ],
        out_specs=pl.BlockSpec((_TILE, _D), lambda i: (i, 0)),
        out_shape=jax.ShapeDtypeStruct((b, _D), jnp.float32),
    )(counts, wf, b1.reshape(1, _D), W2, b2.reshape(1, _D), W3,
      b3.reshape(1, _D))


# speedup vs baseline: 1.5751x; 1.5751x over previous
import functools

import jax
import jax.numpy as jnp
from jax import lax
from jax.experimental import pallas as pl
from jax.experimental.pallas import tpu as pltpu
from jax.experimental.pallas import tpu_sc as plsc

_B = 16384
_L = 200
_D = 256
_NBINS = 48
_NW = 32
_ROWS_PER_W = _B // _NW
_BLK = 64
_NBLK = _ROWS_PER_W // _BLK


def _hist_body(s_hbm, c_hbm, o_hbm, out_hbm, s_v, c_v, o_v, cnt_v):
    wid = lax.axis_index("s") * 2 + lax.axis_index("c")
    base = wid * _ROWS_PER_W
    ones = jnp.ones((16,), jnp.float32)
    zeros = jnp.zeros((16,), jnp.float32)
    lane = lax.iota(jnp.int32, 16)
    tail_mask = lane >= 8

    def blk_body(bi, carry):
        row0 = base + bi * _BLK
        pltpu.sync_copy(s_hbm.at[pl.ds(row0, _BLK)], s_v)
        pltpu.sync_copy(c_hbm.at[pl.ds(row0, _BLK)], c_v)
        pltpu.sync_copy(o_hbm.at[pl.ds(row0, _BLK)], o_v)

        @plsc.parallel_loop(0, _BLK * 3)
        def zero_body(z):
            cnt_v[pl.ds(z * 16, 16)] = zeros

        @plsc.parallel_loop(0, _BLK, unroll=2)
        def row_body(r):
            roff = r * _NBINS
            for ids_v, fbase in ((s_v, 0), (c_v, 20), (o_v, 35)):
                for k in range(12):
                    idx = ids_v[r, pl.ds(k * 16, 16)] + (roff + fbase)
                    plsc.addupdate_scatter(cnt_v, [idx], ones)
                idx = ids_v[r, pl.ds(_L - 16, 16)] + (roff + fbase)
                plsc.addupdate_scatter(cnt_v, [idx], ones, mask=tail_mask)
        pltpu.sync_copy(cnt_v, out_hbm.at[pl.ds(row0 * _NBINS,
                                                _BLK * _NBINS)])
        return carry

    lax.fori_loop(0, _NBLK, blk_body, 0)


@functools.partial(
    pl.kernel,
    out_type=jax.ShapeDtypeStruct((_B * _NBINS,), jnp.float32),
    mesh=plsc.VectorSubcoreMesh(core_axis_name="c", subcore_axis_name="s"),
    compiler_params=pltpu.CompilerParams(needs_layout_passes=False),
    scratch_types=[
        pltpu.VMEM((_BLK, _L), jnp.int32),
        pltpu.VMEM((_BLK, _L), jnp.int32),
        pltpu.VMEM((_BLK, _L), jnp.int32),
        pltpu.VMEM((_BLK * _NBINS,), jnp.float32),
    ],
)
def _sc_histogram(s_hbm, c_hbm, o_hbm, out_hbm, s_v, c_v, o_v, cnt_v):
    _hist_body(s_hbm, c_hbm, o_hbm, out_hbm, s_v, c_v, o_v, cnt_v)


_TILE = 512


def _mlp_kernel_body(cnt_ref, wf_ref, b1_ref, w2_ref, b2_ref, w3_ref, b3_ref,
                     out_ref):
    h = jnp.maximum(
        jnp.dot(cnt_ref[...], wf_ref[...], preferred_element_type=jnp.float32)
        + b1_ref[...], 0.0)
    h = jnp.maximum(
        jnp.dot(h, w2_ref[...], preferred_element_type=jnp.float32)
        + b2_ref[...], 0.0)
    out_ref[...] = (
        jnp.dot(h, w3_ref[...], preferred_element_type=jnp.float32)
        + b3_ref[...])


def kernel(style_ids, color_ids, occasion_ids, style_table, color_table,
           occasion_table, W1, b1, W2, b2, W3, b3):
    b = style_ids.shape[0]
    q = style_table.shape[1]
    wf = jnp.concatenate([
        style_table @ W1[:q],
        color_table @ W1[q:2 * q],
        occasion_table @ W1[2 * q:3 * q],
    ], axis=0) * (1.0 / _L)
    wf = jnp.pad(wf, ((0, _NBINS - wf.shape[0]), (0, 0)))

    counts = _sc_histogram(style_ids, color_ids,
                           occasion_ids).reshape(b, _NBINS)

    grid = (b // _TILE,)
    w_spec = lambda shape: pl.BlockSpec(shape, lambda i: (0,) * len(shape))
    return pl.pallas_call(
        _mlp_kernel_body,
        grid=grid,
        in_specs=[
            pl.BlockSpec((_TILE, _NBINS), lambda i: (i, 0)),
            w_spec((_NBINS, _D)),
            w_spec((1, _D)),
            w_spec((_D, _D)),
            w_spec((1, _D)),
            w_spec((_D, _D)),
            w_spec((1, _D)),
        ],
        out_specs=pl.BlockSpec((_TILE, _D), lambda i: (i, 0)),
        out_shape=jax.ShapeDtypeStruct((b, _D), jnp.float32),
    )(counts, wf, b1.reshape(1, _D), W2, b2.reshape(1, _D), W3,
      b3.reshape(1, _D))


# speedup vs baseline: 1.7931x; 1.1384x over previous
import functools

import jax
import jax.numpy as jnp
from jax import lax
from jax.experimental import pallas as pl
from jax.experimental.pallas import tpu as pltpu
from jax.experimental.pallas import tpu_sc as plsc

_B = 16384
_L = 200
_D = 256
_NBINS = 48
_NW = 32
_ROWS_PER_W = _B // _NW
_BLK = 64
_NBLK = _ROWS_PER_W // _BLK


def _hist_body(s_hbm, c_hbm, o_hbm, out_hbm, bufs, cnts, in_sems, out_sems):
    wid = lax.axis_index("s") * 2 + lax.axis_index("c")
    base = wid * _ROWS_PER_W
    ones = jnp.ones((16,), jnp.float32)
    zeros = jnp.zeros((16,), jnp.float32)
    lane = lax.iota(jnp.int32, 16)
    tail_mask = lane >= 8
    hbms = (s_hbm, c_hbm, o_hbm)

    def in_copies(bi, par):
        row0 = base + bi * _BLK
        return [
            pltpu.make_async_copy(hbm.at[pl.ds(row0, _BLK)], v, in_sems[par])
            for hbm, v in zip(hbms, bufs[par])
        ]

    def out_copy(bi, par):
        row0 = base + bi * _BLK
        return pltpu.make_async_copy(
            cnts[par], out_hbm.at[pl.ds(row0 * _NBINS, _BLK * _NBINS)],
            out_sems[par])

    for cp in in_copies(0, 0):
        cp.start()
    for bi in range(_NBLK):
        par = bi % 2
        if bi + 1 < _NBLK:
            for cp in in_copies(bi + 1, 1 - par):
                cp.start()
        for cp in in_copies(bi, par):
            cp.wait()
        if bi >= 2:
            out_copy(bi - 2, par).wait()
        cnt_v = cnts[par]
        s_v, c_v, o_v = bufs[par]

        @plsc.parallel_loop(0, _BLK * 3)
        def zero_body(z):
            cnt_v[pl.ds(z * 16, 16)] = zeros

        @plsc.parallel_loop(0, _BLK, unroll=2)
        def row_body(r):
            roff = r * _NBINS
            for ids_v, fbase in ((s_v, 0), (c_v, 20), (o_v, 35)):
                for k in range(12):
                    idx = ids_v[r, pl.ds(k * 16, 16)] + (roff + fbase)
                    plsc.addupdate_scatter(cnt_v, [idx], ones)
                idx = ids_v[r, pl.ds(_L - 16, 16)] + (roff + fbase)
                plsc.addupdate_scatter(cnt_v, [idx], ones, mask=tail_mask)

        out_copy(bi, par).start()
    out_copy(_NBLK - 2, 0).wait()
    out_copy(_NBLK - 1, 1).wait()


@functools.partial(
    pl.kernel,
    out_type=jax.ShapeDtypeStruct((_B * _NBINS,), jnp.float32),
    mesh=plsc.VectorSubcoreMesh(core_axis_name="c", subcore_axis_name="s"),
    compiler_params=pltpu.CompilerParams(needs_layout_passes=False),
    scratch_types=[
        pltpu.VMEM((_BLK, _L), jnp.int32),
        pltpu.VMEM((_BLK, _L), jnp.int32),
        pltpu.VMEM((_BLK, _L), jnp.int32),
        pltpu.VMEM((_BLK, _L), jnp.int32),
        pltpu.VMEM((_BLK, _L), jnp.int32),
        pltpu.VMEM((_BLK, _L), jnp.int32),
        pltpu.VMEM((_BLK * _NBINS,), jnp.float32),
        pltpu.VMEM((_BLK * _NBINS,), jnp.float32),
        pltpu.SemaphoreType.DMA,
        pltpu.SemaphoreType.DMA,
        pltpu.SemaphoreType.DMA,
        pltpu.SemaphoreType.DMA,
    ],
)
def _sc_histogram(s_hbm, c_hbm, o_hbm, out_hbm, s0, c0, o0, s1, c1, o1,
                  cnt0, cnt1, isem0, isem1, osem0, osem1):
    _hist_body(s_hbm, c_hbm, o_hbm, out_hbm,
               ((s0, c0, o0), (s1, c1, o1)), (cnt0, cnt1),
               (isem0, isem1), (osem0, osem1))


_TILE = 512


def _mlp_kernel_body(cnt_ref, wf_ref, b1_ref, w2_ref, b2_ref, w3_ref, b3_ref,
                     out_ref):
    h = jnp.maximum(
        jnp.dot(cnt_ref[...], wf_ref[...], preferred_element_type=jnp.float32)
        + b1_ref[...], 0.0)
    h = jnp.maximum(
        jnp.dot(h, w2_ref[...], preferred_element_type=jnp.float32)
        + b2_ref[...], 0.0)
    out_ref[...] = (
        jnp.dot(h, w3_ref[...], preferred_element_type=jnp.float32)
        + b3_ref[...])


def kernel(style_ids, color_ids, occasion_ids, style_table, color_table,
           occasion_table, W1, b1, W2, b2, W3, b3):
    b = style_ids.shape[0]
    q = style_table.shape[1]
    wf = jnp.concatenate([
        style_table @ W1[:q],
        color_table @ W1[q:2 * q],
        occasion_table @ W1[2 * q:3 * q],
    ], axis=0) * (1.0 / _L)
    wf = jnp.pad(wf, ((0, _NBINS - wf.shape[0]), (0, 0)))

    counts = _sc_histogram(style_ids, color_ids,
                           occasion_ids).reshape(b, _NBINS)

    grid = (b // _TILE,)
    w_spec = lambda shape: pl.BlockSpec(shape, lambda i: (0,) * len(shape))
    return pl.pallas_call(
        _mlp_kernel_body,
        grid=grid,
        in_specs=[
            pl.BlockSpec((_TILE, _NBINS), lambda i: (i, 0)),
            w_spec((_NBINS, _D)),
            w_spec((1, _D)),
            w_spec((_D, _D)),
            w_spec((1, _D)),
            w_spec((_D, _D)),
            w_spec((1, _D)),
        ],
        out_specs=pl.BlockSpec((_TILE, _D), lambda i: (i, 0)),
        out_shape=jax.ShapeDtypeStruct((b, _D), jnp.float32),
    )(counts, wf, b1.reshape(1, _D), W2, b2.reshape(1, _D), W3,
      b3.reshape(1, _D))


# speedup vs baseline: 1.8139x; 1.0116x over previous
import functools

import jax
import jax.numpy as jnp
from jax import lax
from jax.experimental import pallas as pl
from jax.experimental.pallas import tpu as pltpu
from jax.experimental.pallas import tpu_sc as plsc

_B = 16384
_L = 200
_D = 256
_NBINS = 48
_NW = 32
_ROWS_PER_W = _B // _NW
_BLK = 64
_NBLK = _ROWS_PER_W // _BLK


def _hist_body(s_hbm, c_hbm, o_hbm, out_hbm, bufs, cnts, in_sems, out_sems):
    wid = lax.axis_index("s") * 2 + lax.axis_index("c")
    base = wid * _ROWS_PER_W
    ones = jnp.ones((16,), jnp.float32)
    zeros = jnp.zeros((16,), jnp.float32)
    lane = lax.iota(jnp.int32, 16)
    tail_mask = lane >= 8
    hbms = (s_hbm, c_hbm, o_hbm)

    def in_copies(row0, par):
        return [
            pltpu.make_async_copy(hbm.at[pl.ds(row0, _BLK)], v, in_sems[par])
            for hbm, v in zip(hbms, bufs[par])
        ]

    def out_copy(row0, par):
        return pltpu.make_async_copy(
            cnts[par], out_hbm.at[pl.ds(row0 * _NBINS, _BLK * _NBINS)],
            out_sems[par])

    for cp in in_copies(base, 0):
        cp.start()
    for cp in in_copies(base + _BLK, 1):
        cp.start()

    def pair_body(i, carry):
        for par in (0, 1):
            row0 = base + (2 * i + par) * _BLK
            for cp in in_copies(row0, par):
                cp.wait()

            @pl.when(i > 0)
            def _wait_out():
                out_copy(row0 - 2 * _BLK, par).wait()

            cnt_v = cnts[par]
            s_v, c_v, o_v = bufs[par]

            @plsc.parallel_loop(0, _BLK * 3)
            def zero_body(z):
                cnt_v[pl.ds(z * 16, 16)] = zeros

            @plsc.parallel_loop(0, _BLK, unroll=4)
            def row_body(r):
                roff = r * _NBINS
                for ids_v, fbase in ((s_v, 0), (c_v, 20), (o_v, 35)):
                    for k in range(12):
                        idx = ids_v[r, pl.ds(k * 16, 16)] + (roff + fbase)
                        plsc.addupdate_scatter(cnt_v, [idx], ones)
                    idx = ids_v[r, pl.ds(_L - 16, 16)] + (roff + fbase)
                    plsc.addupdate_scatter(cnt_v, [idx], ones,
                                           mask=tail_mask)

            out_copy(row0, par).start()

            @pl.when(i < _NBLK // 2 - 1)
            def _next_in():
                for cp in in_copies(row0 + 2 * _BLK, par):
                    cp.start()

        return carry

    lax.fori_loop(0, _NBLK // 2, pair_body, 0)
    out_copy(base + (_NBLK - 2) * _BLK, 0).wait()
    out_copy(base + (_NBLK - 1) * _BLK, 1).wait()


@functools.partial(
    pl.kernel,
    out_type=jax.ShapeDtypeStruct((_B * _NBINS,), jnp.float32),
    mesh=plsc.VectorSubcoreMesh(core_axis_name="c", subcore_axis_name="s"),
    compiler_params=pltpu.CompilerParams(needs_layout_passes=False),
    scratch_types=[
        pltpu.VMEM((_BLK, _L), jnp.int32),
        pltpu.VMEM((_BLK, _L), jnp.int32),
        pltpu.VMEM((_BLK, _L), jnp.int32),
        pltpu.VMEM((_BLK, _L), jnp.int32),
        pltpu.VMEM((_BLK, _L), jnp.int32),
        pltpu.VMEM((_BLK, _L), jnp.int32),
        pltpu.VMEM((_BLK * _NBINS,), jnp.float32),
        pltpu.VMEM((_BLK * _NBINS,), jnp.float32),
        pltpu.SemaphoreType.DMA,
        pltpu.SemaphoreType.DMA,
        pltpu.SemaphoreType.DMA,
        pltpu.SemaphoreType.DMA,
    ],
)
def _sc_histogram(s_hbm, c_hbm, o_hbm, out_hbm, s0, c0, o0, s1, c1, o1,
                  cnt0, cnt1, isem0, isem1, osem0, osem1):
    _hist_body(s_hbm, c_hbm, o_hbm, out_hbm,
               ((s0, c0, o0), (s1, c1, o1)), (cnt0, cnt1),
               (isem0, isem1), (osem0, osem1))


_TILE = 512


def _mlp_kernel_body(cnt_ref, wf_ref, b1_ref, w2_ref, b2_ref, w3_ref, b3_ref,
                     out_ref):
    h = jnp.maximum(
        jnp.dot(cnt_ref[...], wf_ref[...], preferred_element_type=jnp.float32)
        + b1_ref[...], 0.0)
    h = jnp.maximum(
        jnp.dot(h, w2_ref[...], preferred_element_type=jnp.float32)
        + b2_ref[...], 0.0)
    out_ref[...] = (
        jnp.dot(h, w3_ref[...], preferred_element_type=jnp.float32)
        + b3_ref[...])


def kernel(style_ids, color_ids, occasion_ids, style_table, color_table,
           occasion_table, W1, b1, W2, b2, W3, b3):
    b = style_ids.shape[0]
    q = style_table.shape[1]
    wf = jnp.concatenate([
        style_table @ W1[:q],
        color_table @ W1[q:2 * q],
        occasion_table @ W1[2 * q:3 * q],
    ], axis=0) * (1.0 / _L)
    wf = jnp.pad(wf, ((0, _NBINS - wf.shape[0]), (0, 0)))

    counts = _sc_histogram(style_ids, color_ids,
                           occasion_ids).reshape(b, _NBINS)

    grid = (b // _TILE,)
    w_spec = lambda shape: pl.BlockSpec(shape, lambda i: (0,) * len(shape))
    return pl.pallas_call(
        _mlp_kernel_body,
        grid=grid,
        in_specs=[
            pl.BlockSpec((_TILE, _NBINS), lambda i: (i, 0)),
            w_spec((_NBINS, _D)),
            w_spec((1, _D)),
            w_spec((_D, _D)),
            w_spec((1, _D)),
            w_spec((_D, _D)),
            w_spec((1, _D)),
        ],
        out_specs=pl.BlockSpec((_TILE, _D), lambda i: (i, 0)),
        out_shape=jax.ShapeDtypeStruct((b, _D), jnp.float32),
    )(counts, wf, b1.reshape(1, _D), W2, b2.reshape(1, _D), W3,
      b3.reshape(1, _D))
